# Initial kernel scaffold; baseline (speedup 1.0000x reference)
import jax
import jax.numpy as jnp
from jax.experimental import pallas as pl


def kernel(x, edge_index, edge_weight, W, b, k):
    raise NotImplementedError("write your pallas kernel here")


# speedup vs baseline: 3.0045x; 3.0045x over previous
import functools

import jax
import jax.numpy as jnp
from jax import lax
from jax.experimental import pallas as pl
from jax.experimental.pallas import tpu as pltpu
from jax.experimental.pallas import tpu_sc as plsc

N = 10000
E = 320000
D = 128

NC = 2
NS = 16
NW = NC * NS
LANES = 16

CHUNK = 128
CHUNKS_TOTAL = -(-E // CHUNK)
CPW = -(-CHUNKS_TOTAL // NW)
CHUNKS_PAD = CPW * NW
E_PAD = CHUNKS_PAD * CHUNK
RPT = 8 * (-(-N // (8 * NS)))
N_PAD = RPT * NS

_mesh = plsc.VectorSubcoreMesh(
    core_axis_name="c", subcore_axis_name="s", num_cores=NC, num_subcores=NS)


@functools.partial(
    pl.kernel,
    out_type=jax.ShapeDtypeStruct((NC, N_PAD, D), jnp.float32),
    mesh=_mesh,
    scratch_types=[
        pltpu.VMEM((CHUNK, D), jnp.float32),
        pltpu.VMEM((CHUNK,), jnp.int32),
        pltpu.VMEM((CHUNK,), jnp.int32),
        pltpu.VMEM((CHUNK, LANES), jnp.float32),
        pltpu.VMEM_SHARED((N_PAD, D), jnp.float32),
        pltpu.SemaphoreType.DMA,
    ],
)
def _spmm_sc(h_hbm, zeros_hbm, col_hbm, row_hbm, w_hbm, out_hbm,
             rows_v, col_v, row_v, w_v, acc_sh, sem):
    c = lax.axis_index("c")
    s = lax.axis_index("s")
    wid = s * NC + c

    pltpu.sync_copy(zeros_hbm.at[pl.ds(s * RPT, RPT)],
                    acc_sh.at[pl.ds(s * RPT, RPT)])
    plsc.subcore_barrier()

    def chunk_body(j, carry):
        base = wid * CPW + j
        pltpu.sync_copy(col_hbm.at[base], col_v)
        pltpu.sync_copy(row_hbm.at[base], row_v)
        pltpu.sync_copy(w_hbm.at[base], w_v)
        pltpu.async_copy(h_hbm.at[col_v], rows_v, sem).wait()

        def edge_body(i, carry2):
            wv = w_v[i, :]
            for jj in range(D // LANES):
                sl = (i, pl.ds(jj * LANES, LANES))
                rows_v[sl] = rows_v[sl] * wv
            return carry2
        lax.fori_loop(0, CHUNK, edge_body, 0, unroll=False)

        pltpu.sync_copy(rows_v, acc_sh.at[row_v], add=True)
        return carry

    lax.fori_loop(0, CPW, chunk_body, 0, unroll=False)
    plsc.subcore_barrier()

    pltpu.sync_copy(acc_sh.at[pl.ds(s * RPT, RPT)],
                    out_hbm.at[c, pl.ds(s * RPT, RPT)])


_BN = 1000
_BC = RPT


def _combine_tc(p):
    def body(p_ref, o_ref):
        o_ref[...] = p_ref[0] + p_ref[1]
    return pl.pallas_call(
        body,
        grid=(N_PAD // _BC,),
        in_specs=[pl.BlockSpec((2, _BC, D), lambda i: (0, i, 0))],
        out_specs=pl.BlockSpec((_BC, D), lambda i: (i, 0)),
        out_shape=jax.ShapeDtypeStruct((N_PAD, D), jnp.float32),
    )(p)


def _linear_tc(h, W, b2):
    def body(h_ref, w_ref, b_ref, o_ref):
        acc = lax.dot_general(h_ref[...], w_ref[...],
                              (((1,), (1,)), ((), ())),
                              preferred_element_type=jnp.float32)
        o_ref[...] = acc + b_ref[...]
    return pl.pallas_call(
        body,
        grid=(N // _BN,),
        in_specs=[
            pl.BlockSpec((_BN, D), lambda i: (i, 0)),
            pl.BlockSpec((D, D), lambda i: (0, 0)),
            pl.BlockSpec((1, D), lambda i: (0, 0)),
        ],
        out_specs=pl.BlockSpec((_BN, D), lambda i: (i, 0)),
        out_shape=jax.ShapeDtypeStruct((N, D), jnp.float32),
    )(h, W, b2)


def kernel(x, edge_index, edge_weight, W, b, k):
    row = edge_index[0]
    col = edge_index[1]
    pad = E_PAD - E
    col2 = jnp.pad(col, (0, pad)).reshape(CHUNKS_PAD, CHUNK)
    row2 = jnp.pad(row, (0, pad)).reshape(CHUNKS_PAD, CHUNK)
    w2 = jnp.broadcast_to(
        jnp.pad(edge_weight, (0, pad)).reshape(CHUNKS_PAD, CHUNK, 1),
        (CHUNKS_PAD, CHUNK, LANES)).astype(jnp.float32)
    zeros = jnp.zeros((N_PAD, D), jnp.float32)
    b2 = b.reshape(1, D)
    x_pad = jnp.pad(x, ((0, N_PAD - N), (0, 0)))

    def it_body(_, h):
        p = _spmm_sc(h, zeros, col2, row2, w2)
        return _combine_tc(p)

    h = lax.fori_loop(0, k, it_body, x_pad)
    return _linear_tc(h[:N], W, b2)


# speedup vs baseline: 3.8859x; 1.2933x over previous
import functools

import jax
import jax.numpy as jnp
from jax import lax
from jax.experimental import pallas as pl
from jax.experimental.pallas import tpu as pltpu
from jax.experimental.pallas import tpu_sc as plsc

N = 10000
E = 320000
D = 128

NC = 2
NS = 16
NW = NC * NS
LANES = 16

CHUNK = 128
CHUNKS_TOTAL = -(-E // CHUNK)
CPW = 2 * (-(-CHUNKS_TOTAL // (2 * NW)))
CHUNKS_PAD = CPW * NW
E_PAD = CHUNKS_PAD * CHUNK
MROWS = 2 + CHUNK * LANES // 128
RPT = 8 * (-(-N // (8 * NS)))
N_PAD = RPT * NS

_mesh = plsc.VectorSubcoreMesh(
    core_axis_name="c", subcore_axis_name="s", num_cores=NC, num_subcores=NS)


@functools.partial(
    pl.kernel,
    out_type=jax.ShapeDtypeStruct((NC, N_PAD, D), jnp.float32),
    mesh=_mesh,
    scratch_types=[
        pltpu.VMEM((2, CHUNK, D), jnp.float32),
        pltpu.VMEM((2, 2, CHUNK), jnp.int32),
        pltpu.VMEM((2, LANES, CHUNK), jnp.float32),
        pltpu.VMEM_SHARED((N_PAD, D), jnp.float32),
        pltpu.SemaphoreType.DMA,
        pltpu.SemaphoreType.DMA,
    ],
)
def _spmm_sc(h_hbm, zeros_hbm, idx_hbm, w_hbm, out_hbm,
             rows_v, idx_v, w_v, acc_sh, sg0, sg1):
    c = lax.axis_index("c")
    s = lax.axis_index("s")
    wid = s * NC + c
    sg = (sg0, sg1)

    pltpu.sync_copy(zeros_hbm.at[pl.ds(s * RPT, RPT)],
                    acc_sh.at[pl.ds(s * RPT, RPT)])
    plsc.subcore_barrier()

    base = wid * CPW
    pltpu.sync_copy(idx_hbm.at[base], idx_v.at[0])
    pltpu.sync_copy(w_hbm.at[base], w_v.at[0])
    pltpu.async_copy(h_hbm.at[idx_v.at[0, 0]], rows_v.at[0], sg[0])

    def step(j, b):
        nb = 1 - b
        @pl.when(j + 1 < CPW)
        def _():
            pltpu.sync_copy(idx_hbm.at[base + j + 1], idx_v.at[nb])
            pltpu.sync_copy(w_hbm.at[base + j + 1], w_v.at[nb])
            pltpu.async_copy(h_hbm.at[idx_v.at[nb, 0]], rows_v.at[nb], sg[nb])

        pltpu.make_async_copy(h_hbm.at[idx_v.at[b, 0]], rows_v.at[b],
                              sg[b]).wait()

        def edge_body(i, carry):
            wv = w_v[b, i // 8, pl.ds((i % 8) * LANES, LANES)]
            for jj in range(D // LANES):
                sl = (b, i, pl.ds(jj * LANES, LANES))
                rows_v[sl] = rows_v[sl] * wv
            return carry
        lax.fori_loop(0, CHUNK, edge_body, 0, unroll=4)

        pltpu.sync_copy(rows_v.at[b], acc_sh.at[idx_v.at[b, 1]], add=True)

    def loop_body(jj, carry):
        step(2 * jj, 0)
        step(2 * jj + 1, 1)
        return carry
    lax.fori_loop(0, CPW // 2, loop_body, 0, unroll=False)

    plsc.subcore_barrier()
    pltpu.sync_copy(acc_sh.at[pl.ds(s * RPT, RPT)],
                    out_hbm.at[c, pl.ds(s * RPT, RPT)])


_BN = 1000
_BC = RPT


def _combine_tc(p):
    def body(p_ref, o_ref):
        o_ref[...] = p_ref[0] + p_ref[1]
    return pl.pallas_call(
        body,
        grid=(N_PAD // _BC,),
        in_specs=[pl.BlockSpec((2, _BC, D), lambda i: (0, i, 0))],
        out_specs=pl.BlockSpec((_BC, D), lambda i: (i, 0)),
        out_shape=jax.ShapeDtypeStruct((N_PAD, D), jnp.float32),
    )(p)


def _linear_tc(h, W, b2):
    def body(h_ref, w_ref, b_ref, o_ref):
        acc = lax.dot_general(h_ref[...], w_ref[...],
                              (((1,), (1,)), ((), ())),
                              preferred_element_type=jnp.float32)
        o_ref[...] = acc + b_ref[...]
    return pl.pallas_call(
        body,
        grid=(N // _BN,),
        in_specs=[
            pl.BlockSpec((_BN, D), lambda i: (i, 0)),
            pl.BlockSpec((D, D), lambda i: (0, 0)),
            pl.BlockSpec((1, D), lambda i: (0, 0)),
        ],
        out_specs=pl.BlockSpec((_BN, D), lambda i: (i, 0)),
        out_shape=jax.ShapeDtypeStruct((N, D), jnp.float32),
    )(h, W, b2)


def kernel(x, edge_index, edge_weight, W, b, k):
    row = edge_index[0]
    col = edge_index[1]
    pad = E_PAD - E
    col2 = jnp.pad(col, (0, pad)).reshape(CHUNKS_PAD, 1, CHUNK)
    row2 = jnp.pad(row, (0, pad)).reshape(CHUNKS_PAD, 1, CHUNK)
    idx = jnp.concatenate([col2, row2], axis=1)
    w2 = jnp.broadcast_to(
        jnp.pad(edge_weight, (0, pad)).reshape(CHUNKS_PAD, CHUNK, 1),
        (CHUNKS_PAD, CHUNK, LANES)).reshape(CHUNKS_PAD, LANES, CHUNK)
    zeros = jnp.zeros((N_PAD, D), jnp.float32)
    b2 = b.reshape(1, D)
    x_pad = jnp.pad(x, ((0, N_PAD - N), (0, 0)))

    def it_body(_, h):
        p = _spmm_sc(h, zeros, idx, w2)
        return _combine_tc(p)

    h = lax.fori_loop(0, k, it_body, x_pad)
    return _linear_tc(h[:N], W, b2)
